# baseline (speedup 1.0000x reference)
import functools

import jax
import jax.numpy as jnp
from jax import lax
from jax.experimental import pallas as pl
from jax.experimental.pallas import tpu as pltpu
from jax.experimental.pallas import tpu_sc as plsc

B, L = 4096, 50
N = B * L
D_ITEM, D_CAT, D_USER = 64, 32, 64
D_OUT = D_ITEM + D_CAT + D_USER
DT = D_OUT // 8
NUM_CORES = 2
NUM_SUBCORES = 16
NW = NUM_CORES * NUM_SUBCORES
BBLK = B // NW
NBUF = 2
LANES = 16
NG = BBLK // LANES


def _body(item_h, cat_h, user_h, wi_h, wc_h, wu_h, out_h,
          idxt_i, idxt_c, idxt_u,
          ri0, rc0, ru0, ri1, rc1, ru1, t0, t1,
          gsem0, gsem1, wsem0, wsem1):
    wid = lax.axis_index("s") * NUM_CORES + lax.axis_index("c")
    rows = ((ri0, rc0, ru0), (ri1, rc1, ru1))
    tbufs = (t0, t1)
    gsems = (gsem0, gsem1)
    wsems = (wsem0, wsem1)

    col = pl.ds(wid * BBLK, BBLK)
    pltpu.sync_copy(item_h.at[:, col], idxt_i)
    pltpu.sync_copy(cat_h.at[:, col], idxt_c)
    pltpu.sync_copy(user_h.at[:, col], idxt_u)

    iota16 = lax.iota(jnp.int32, LANES)

    def gather_copies(l, s):
        ri, rc, ru = rows[s]
        sem = gsems[s]
        return (
            pltpu.make_async_copy(wi_h.at[idxt_i.at[l]], ri, sem),
            pltpu.make_async_copy(wc_h.at[idxt_c.at[l]], rc, sem),
            pltpu.make_async_copy(wu_h.at[idxt_u.at[l]], ru, sem),
        )

    def write_copies(l, s):
        return (
            pltpu.make_async_copy(
                tbufs[s].at[:, :, pl.ds(0, BBLK)],
                out_h.at[l, pl.ds(0, DT), wid], wsems[s]),
        )

    for cp in gather_copies(0, 0):
        cp.start()

    def step(l, s):
        other = 1 - s

        @pl.when(jnp.logical_and(l >= 1, l + 1 < L))
        def _():
            for cp in write_copies(l - 1, other):
                cp.wait()

        @pl.when(l + 1 < L)
        def _():
            for cp in gather_copies(l + 1, other):
                cp.start()

        for cp in gather_copies(l, s):
            cp.wait()

        ri, rc, ru = rows[s]
        tb = tbufs[s]

        @plsc.parallel_loop(0, NG, unroll=2)
        def _(g):
            g16 = g * LANES
            mz16 = tuple(
                jnp.where(idxt[l, pl.ds(g16, LANES)] == 0, 0.0, 1.0)
                for idxt in (idxt_i, idxt_c, idxt_u))
            for i in range(LANES):
                b = g16 + i
                bv = jnp.full((LANES,), b, jnp.int32)
                for t, (src, d, base_d) in enumerate((
                        (ri, D_ITEM, 0),
                        (rc, D_CAT, D_ITEM),
                        (ru, D_USER, D_ITEM + D_CAT))):
                    mz = mz16[t][i]
                    for d0 in range(0, d, LANES):
                        v = src[b, pl.ds(d0, LANES)] * mz
                        dfv = iota16 + (base_d + d0)
                        plsc.store_scatter(tb, [dfv // 8, dfv % 8, bv], v)

        for cp in write_copies(l, s):
            cp.start()

    def outer_body(lo, carry):
        for b in range(NBUF):
            step(lo * NBUF + b, b)
        return carry

    lax.fori_loop(0, L // NBUF, outer_body, 0)

    for cp in write_copies(L - 2, (L - 2) % NBUF):
        cp.wait()
    for cp in write_copies(L - 1, (L - 1) % NBUF):
        cp.wait()


@jax.jit
def _run(item_t, cat_t, user_t, W_item, W_category, W_user):
    mesh = plsc.VectorSubcoreMesh(core_axis_name="c", subcore_axis_name="s")
    k = functools.partial(
        pl.kernel,
        mesh=mesh,
        compiler_params=pltpu.CompilerParams(
            use_tc_tiling_on_sc=False, needs_layout_passes=False),
        out_type=jax.ShapeDtypeStruct((L, DT, NW, 8, BBLK), jnp.float32),
        scratch_types=[
            pltpu.VMEM((L, BBLK), jnp.int32),
            pltpu.VMEM((L, BBLK), jnp.int32),
            pltpu.VMEM((L, BBLK), jnp.int32),
            pltpu.VMEM((BBLK, D_ITEM), jnp.float32),
            pltpu.VMEM((BBLK, D_CAT), jnp.float32),
            pltpu.VMEM((BBLK, D_USER), jnp.float32),
            pltpu.VMEM((BBLK, D_ITEM), jnp.float32),
            pltpu.VMEM((BBLK, D_CAT), jnp.float32),
            pltpu.VMEM((BBLK, D_USER), jnp.float32),
            pltpu.VMEM((DT, 8, BBLK + 1), jnp.float32),
            pltpu.VMEM((DT, 8, BBLK + 1), jnp.float32),
            pltpu.SemaphoreType.DMA,
            pltpu.SemaphoreType.DMA,
            pltpu.SemaphoreType.DMA,
            pltpu.SemaphoreType.DMA,
        ],
    )(_body)
    return k(item_t, cat_t, user_t, W_item, W_category, W_user)


def kernel(item, category, user, W_item, W_category, W_user):
    item_t = item.astype(jnp.int32).T
    cat_t = category.astype(jnp.int32).T
    user_t = user.astype(jnp.int32).T
    out5 = _run(item_t, cat_t, user_t, W_item, W_category, W_user)
    return out5.transpose(2, 4, 0, 1, 3).reshape(B, L, D_OUT)


# speedup vs baseline: 1.0435x; 1.0435x over previous
import functools

import jax
import jax.numpy as jnp
from jax import lax
from jax.experimental import pallas as pl
from jax.experimental.pallas import tpu as pltpu
from jax.experimental.pallas import tpu_sc as plsc

B, L = 4096, 50
N = B * L
D_ITEM, D_CAT, D_USER = 64, 32, 64
D_OUT = D_ITEM + D_CAT + D_USER
DT = D_OUT // 8
NUM_CORES = 2
NUM_SUBCORES = 16
NW = NUM_CORES * NUM_SUBCORES
BBLK = B // NW
NBUF = 2
LANES = 16
NG = BBLK // LANES


def _body(item_h, cat_h, user_h, wi_h, wc_h, wu_h, out_h,
          idxt_i, idxt_c, idxt_u,
          ri0, rc0, ru0, ri1, rc1, ru1, t0, t1,
          gsem0, gsem1, wsem0, wsem1):
    wid = lax.axis_index("s") * NUM_CORES + lax.axis_index("c")
    rows = ((ri0, rc0, ru0), (ri1, rc1, ru1))
    tbufs = (t0, t1)
    gsems = (gsem0, gsem1)
    wsems = (wsem0, wsem1)

    col = pl.ds(wid * BBLK, BBLK)
    pltpu.sync_copy(item_h.at[:, col], idxt_i)
    pltpu.sync_copy(cat_h.at[:, col], idxt_c)
    pltpu.sync_copy(user_h.at[:, col], idxt_u)

    iota16 = lax.iota(jnp.int32, LANES)

    def gather_copies(l, s):
        ri, rc, ru = rows[s]
        sem = gsems[s]
        return (
            pltpu.make_async_copy(wi_h.at[idxt_i.at[l]], ri, sem),
            pltpu.make_async_copy(wc_h.at[idxt_c.at[l]], rc, sem),
            pltpu.make_async_copy(wu_h.at[idxt_u.at[l]], ru, sem),
        )

    def write_copies(l, s):
        return (
            pltpu.make_async_copy(
                tbufs[s].at[:, :, pl.ds(0, BBLK)],
                out_h.at[l, pl.ds(0, DT), wid], wsems[s]),
        )

    for cp in gather_copies(0, 0):
        cp.start()

    def step(l, s):
        other = 1 - s

        @pl.when(jnp.logical_and(l >= 1, l + 1 < L))
        def _():
            for cp in write_copies(l - 1, other):
                cp.wait()

        @pl.when(l + 1 < L)
        def _():
            for cp in gather_copies(l + 1, other):
                cp.start()

        for cp in gather_copies(l, s):
            cp.wait()

        ri, rc, ru = rows[s]
        tb = tbufs[s]

        @plsc.parallel_loop(0, NG, unroll=1)
        def _(g):
            g16 = g * LANES
            mz16 = tuple(
                jnp.where(idxt[l, pl.ds(g16, LANES)] == 0, 0.0, 1.0)
                for idxt in (idxt_i, idxt_c, idxt_u))
            for i in range(LANES):
                b = g16 + i
                bv = jnp.full((LANES,), b, jnp.int32)
                for t, (src, d, base_d) in enumerate((
                        (ri, D_ITEM, 0),
                        (rc, D_CAT, D_ITEM),
                        (ru, D_USER, D_ITEM + D_CAT))):
                    mz = mz16[t][i]
                    for d0 in range(0, d, LANES):
                        v = src[b, pl.ds(d0, LANES)] * mz
                        dfv = iota16 + (base_d + d0)
                        plsc.store_scatter(tb, [dfv // 8, dfv % 8, bv], v)

        for cp in write_copies(l, s):
            cp.start()

    def outer_body(lo, carry):
        for b in range(NBUF):
            step(lo * NBUF + b, b)
        return carry

    lax.fori_loop(0, L // NBUF, outer_body, 0)

    for cp in write_copies(L - 2, (L - 2) % NBUF):
        cp.wait()
    for cp in write_copies(L - 1, (L - 1) % NBUF):
        cp.wait()


@jax.jit
def _run(item_t, cat_t, user_t, W_item, W_category, W_user):
    mesh = plsc.VectorSubcoreMesh(core_axis_name="c", subcore_axis_name="s")
    k = functools.partial(
        pl.kernel,
        mesh=mesh,
        compiler_params=pltpu.CompilerParams(
            use_tc_tiling_on_sc=False, needs_layout_passes=False),
        out_type=jax.ShapeDtypeStruct((L, DT, NW, 8, BBLK), jnp.float32),
        scratch_types=[
            pltpu.VMEM((L, BBLK), jnp.int32),
            pltpu.VMEM((L, BBLK), jnp.int32),
            pltpu.VMEM((L, BBLK), jnp.int32),
            pltpu.VMEM((BBLK, D_ITEM), jnp.float32),
            pltpu.VMEM((BBLK, D_CAT), jnp.float32),
            pltpu.VMEM((BBLK, D_USER), jnp.float32),
            pltpu.VMEM((BBLK, D_ITEM), jnp.float32),
            pltpu.VMEM((BBLK, D_CAT), jnp.float32),
            pltpu.VMEM((BBLK, D_USER), jnp.float32),
            pltpu.VMEM((DT, 8, BBLK + 1), jnp.float32),
            pltpu.VMEM((DT, 8, BBLK + 1), jnp.float32),
            pltpu.SemaphoreType.DMA,
            pltpu.SemaphoreType.DMA,
            pltpu.SemaphoreType.DMA,
            pltpu.SemaphoreType.DMA,
        ],
    )(_body)
    return k(item_t, cat_t, user_t, W_item, W_category, W_user)


def kernel(item, category, user, W_item, W_category, W_user):
    item_t = item.astype(jnp.int32).T
    cat_t = category.astype(jnp.int32).T
    user_t = user.astype(jnp.int32).T
    out5 = _run(item_t, cat_t, user_t, W_item, W_category, W_user)
    return out5.transpose(2, 4, 0, 1, 3).reshape(B, L, D_OUT)


# speedup vs baseline: 1.1804x; 1.1312x over previous
import functools

import jax
import jax.numpy as jnp
from jax import lax
from jax.experimental import pallas as pl
from jax.experimental.pallas import tpu as pltpu
from jax.experimental.pallas import tpu_sc as plsc

B, L = 4096, 50
N = B * L
D_ITEM, D_CAT, D_USER = 64, 32, 64
D_OUT = D_ITEM + D_CAT + D_USER
DT = D_OUT // 8
NUM_CORES = 2
NUM_SUBCORES = 16
NW = NUM_CORES * NUM_SUBCORES
BBLK = B // NW
NBUF = 2
LANES = 16
NG = BBLK // LANES


def _body(item_h, cat_h, user_h, wi_h, wc_h, wu_h, out_h,
          idxt_i, idxt_c, idxt_u,
          ri0, rc0, ru0, ri1, rc1, ru1, t0, t1,
          gsem0, gsem1, wsem0, wsem1):
    wid = lax.axis_index("s") * NUM_CORES + lax.axis_index("c")
    rows = ((ri0, rc0, ru0), (ri1, rc1, ru1))
    tbufs = (t0, t1)
    gsems = (gsem0, gsem1)
    wsems = (wsem0, wsem1)

    col = pl.ds(wid * BBLK, BBLK)
    pltpu.sync_copy(item_h.at[:, col], idxt_i)
    pltpu.sync_copy(cat_h.at[:, col], idxt_c)
    pltpu.sync_copy(user_h.at[:, col], idxt_u)

    iota16 = lax.iota(jnp.int32, LANES)

    def gather_copies(l, s):
        ri, rc, ru = rows[s]
        sem = gsems[s]
        return (
            pltpu.make_async_copy(wi_h.at[idxt_i.at[l]], ri, sem),
            pltpu.make_async_copy(wc_h.at[idxt_c.at[l]], rc, sem),
            pltpu.make_async_copy(wu_h.at[idxt_u.at[l]], ru, sem),
        )

    def write_copies(l, s):
        return (
            pltpu.make_async_copy(
                tbufs[s].at[:, :, pl.ds(0, BBLK)],
                out_h.at[l, pl.ds(0, DT), wid], wsems[s]),
        )

    for cp in gather_copies(0, 0):
        cp.start()

    def step(l, s):
        other = 1 - s

        @pl.when(jnp.logical_and(l >= 1, l + 1 < L))
        def _():
            for cp in write_copies(l - 1, other):
                cp.wait()

        @pl.when(l + 1 < L)
        def _():
            for cp in gather_copies(l + 1, other):
                cp.start()

        for cp in gather_copies(l, s):
            cp.wait()

        ri, rc, ru = rows[s]
        tb = tbufs[s]

        @plsc.parallel_loop(0, NG, unroll=1)
        def _(g):
            g16 = g * LANES
            for i in range(LANES):
                b = g16 + i
                bv = jnp.full((LANES,), b, jnp.int32)
                for src, d, base_d in (
                        (ri, D_ITEM, 0),
                        (rc, D_CAT, D_ITEM),
                        (ru, D_USER, D_ITEM + D_CAT)):
                    for d0 in range(0, d, LANES):
                        v = src[b, pl.ds(d0, LANES)]
                        dfv = iota16 + (base_d + d0)
                        plsc.store_scatter(tb, [dfv // 8, dfv % 8, bv], v)

            bslice = pl.ds(g16, LANES)
            for idxt, d, base_d in (
                    (idxt_i, D_ITEM, 0),
                    (idxt_c, D_CAT, D_ITEM),
                    (idxt_u, D_USER, D_ITEM + D_CAT)):
                iv = idxt[l, bslice]

                @pl.when(jnp.min(iv) == 0)
                def _():
                    mzv = jnp.where(iv == 0, 0.0, 1.0)
                    for dd in range(d):
                        dfull = base_d + dd
                        tb[dfull // 8, dfull % 8, bslice] = (
                            tb[dfull // 8, dfull % 8, bslice] * mzv)

        for cp in write_copies(l, s):
            cp.start()

    def outer_body(lo, carry):
        for b in range(NBUF):
            step(lo * NBUF + b, b)
        return carry

    lax.fori_loop(0, L // NBUF, outer_body, 0)

    for cp in write_copies(L - 2, (L - 2) % NBUF):
        cp.wait()
    for cp in write_copies(L - 1, (L - 1) % NBUF):
        cp.wait()


@jax.jit
def _run(item_t, cat_t, user_t, W_item, W_category, W_user):
    mesh = plsc.VectorSubcoreMesh(core_axis_name="c", subcore_axis_name="s")
    k = functools.partial(
        pl.kernel,
        mesh=mesh,
        compiler_params=pltpu.CompilerParams(
            use_tc_tiling_on_sc=False, needs_layout_passes=False),
        out_type=jax.ShapeDtypeStruct((L, DT, NW, 8, BBLK), jnp.float32),
        scratch_types=[
            pltpu.VMEM((L, BBLK), jnp.int32),
            pltpu.VMEM((L, BBLK), jnp.int32),
            pltpu.VMEM((L, BBLK), jnp.int32),
            pltpu.VMEM((BBLK, D_ITEM), jnp.float32),
            pltpu.VMEM((BBLK, D_CAT), jnp.float32),
            pltpu.VMEM((BBLK, D_USER), jnp.float32),
            pltpu.VMEM((BBLK, D_ITEM), jnp.float32),
            pltpu.VMEM((BBLK, D_CAT), jnp.float32),
            pltpu.VMEM((BBLK, D_USER), jnp.float32),
            pltpu.VMEM((DT, 8, BBLK + 1), jnp.float32),
            pltpu.VMEM((DT, 8, BBLK + 1), jnp.float32),
            pltpu.SemaphoreType.DMA,
            pltpu.SemaphoreType.DMA,
            pltpu.SemaphoreType.DMA,
            pltpu.SemaphoreType.DMA,
        ],
    )(_body)
    return k(item_t, cat_t, user_t, W_item, W_category, W_user)


def kernel(item, category, user, W_item, W_category, W_user):
    item_t = item.astype(jnp.int32).T
    cat_t = category.astype(jnp.int32).T
    user_t = user.astype(jnp.int32).T
    out5 = _run(item_t, cat_t, user_t, W_item, W_category, W_user)
    return out5.transpose(2, 4, 0, 1, 3).reshape(B, L, D_OUT)
